# baseline (speedup 1.0000x reference)
import functools

import jax
import jax.numpy as jnp
from jax import lax
from jax.experimental import pallas as pl
from jax.experimental.pallas import tpu as pltpu
from jax.experimental.pallas import tpu_sc as plsc

_CHUNK = 128
_SG = 8
_NBUF = 3
_NW = 32


def _make_gather(dim: int, n: int):
    tok_w = n // _NW
    chunks = tok_w // _CHUNK
    ngroups = chunks // _SG
    rows_per_g = _SG * _CHUNK
    mesh = plsc.VectorSubcoreMesh(core_axis_name="c", subcore_axis_name="s")

    @functools.partial(
        pl.kernel,
        mesh=mesh,
        out_type=jax.ShapeDtypeStruct((n, dim), jnp.float32),
        compiler_params=pltpu.CompilerParams(use_tc_tiling_on_sc=False),
        scratch_types=[
            pltpu.VMEM((tok_w,), jnp.int32),
            *[pltpu.VMEM((rows_per_g, dim), jnp.float32) for _ in range(_NBUF)],
            *[pltpu.SemaphoreType.DMA for _ in range(2 * _NBUF)],
        ],
    )
    def gather_kernel(idx_hbm, table_hbm, out_hbm, idx_v, *scr):
        stage = scr[:_NBUF]
        gsem = scr[_NBUF:2 * _NBUF]
        wsem = scr[2 * _NBUF:]
        wid = lax.axis_index("s") * 2 + lax.axis_index("c")
        base = wid * tok_w
        pltpu.sync_copy(idx_hbm.at[pl.ds(base, tok_w)], idx_v)

        def fire(g, p):
            return [
                pltpu.async_copy(
                    table_hbm.at[idx_v.at[pl.ds((g * _SG + i) * _CHUNK, _CHUNK)]],
                    stage[p].at[pl.ds(i * _CHUNK, _CHUNK)],
                    gsem[p],
                )
                for i in range(_SG)
            ]

        pending = [None] * _NBUF
        writes = [None] * _NBUF
        pending[0] = fire(0, 0)
        for g in range(ngroups):
            p = g % _NBUF
            if g + 1 < ngroups:
                q = (g + 1) % _NBUF
                if writes[q] is not None:
                    writes[q].wait()
                    writes[q] = None
                pending[q] = fire(g + 1, q)
            for c in pending[p]:
                c.wait()
            writes[p] = pltpu.async_copy(
                stage[p],
                out_hbm.at[pl.ds(base + g * rows_per_g, rows_per_g)],
                wsem[p],
            )
        for w in writes:
            if w is not None:
                w.wait()

    return gather_kernel


def kernel(token_ids, weight):
    b, s = token_ids.shape
    _, dim = weight.shape
    n = b * s
    ids = token_ids.astype(jnp.int32).reshape(n)
    out = _make_gather(dim, n)(ids, weight)
    return out.reshape(b, s, dim)
